# baseline (speedup 1.0000x reference)
import functools
import jax
import jax.numpy as jnp
from jax import lax
from jax.experimental import pallas as pl
from jax.experimental.pallas import tpu as pltpu
from jax.experimental.pallas import tpu_sc as plsc

BATCH = 16384
LATENT = 16
NUM_ROWS = 1000000
BLOCKS = NUM_ROWS // LATENT
NC = 2
NS = 16
NW = NC * NS
BPW = BATCH // NW
CHUNK = 128
CHUNKS = BPW // CHUNK
SUBGROUPS = CHUNK // LATENT


def _gmf_body(uidx_hbm, iidx_hbm, gu_hbm, gi_hbm, wvec_hbm, bvec_hbm, out_hbm,
              sidx_u, sidx_i, ublk_v, iblk_v, ubuf, ibuf,
              wvec_v, bvec_v, logits_v, sem):
    wid = lax.axis_index("s") * NC + lax.axis_index("c")
    base = wid * BPW

    for j in range(CHUNKS):
        pltpu.sync_copy(uidx_hbm.at[pl.ds(base + j * CHUNK, CHUNK)], sidx_u.at[j])
        pltpu.sync_copy(iidx_hbm.at[pl.ds(base + j * CHUNK, CHUNK)], sidx_i.at[j])
    pltpu.sync_copy(wvec_hbm, wvec_v)
    pltpu.sync_copy(bvec_hbm, bvec_v)

    iota = lax.iota(jnp.int32, LATENT)

    def blk_body(t, carry):
        j = t // SUBGROUPS
        s0 = (t % SUBGROUPS) * LATENT
        ublk_v[j, pl.ds(s0, LATENT)] = sidx_u[j, pl.ds(s0, LATENT)] >> 4
        iblk_v[j, pl.ds(s0, LATENT)] = sidx_i[j, pl.ds(s0, LATENT)] >> 4
        return carry

    lax.fori_loop(0, CHUNKS * SUBGROUPS, blk_body, 0)

    bv = bvec_v[...]
    wv = wvec_v[...]
    dsplats = [jnp.full((LATENT,), d, jnp.int32) for d in range(LATENT)]

    def chunk_body(j, carry):
        copies = []
        for d in range(LATENT):
            copies.append(pltpu.async_copy(
                gu_hbm.at[d].at[ublk_v.at[j]], ubuf.at[d], sem))
            copies.append(pltpu.async_copy(
                gi_hbm.at[d].at[iblk_v.at[j]], ibuf.at[d], sem))
        for c in copies:
            c.wait()

        def group_body(g, carry2):
            s0 = g * LATENT
            lane_u = sidx_u[j, pl.ds(s0, LATENT)] & 15
            lane_i = sidx_i[j, pl.ds(s0, LATENT)] & 15
            rows16 = s0 + iota
            acc = bv
            for d in range(LATENT):
                u = plsc.load_gather(ubuf, [dsplats[d], rows16, lane_u])
                i = plsc.load_gather(ibuf, [dsplats[d], rows16, lane_i])
                acc = acc + u * i * wv[d]
            logits_v[pl.ds(j * CHUNK + s0, LATENT)] = 1.0 / (1.0 + jnp.exp(-acc))
            return carry2

        lax.fori_loop(0, SUBGROUPS, group_body, 0)
        return carry

    lax.fori_loop(0, CHUNKS, chunk_body, 0)

    pltpu.sync_copy(logits_v, out_hbm.at[pl.ds(base, BPW)])


def kernel(user_indices, item_indices, domain_idc, embedding_user,
           embedding_item, affine_W, affine_b):
    del domain_idc
    uidx = user_indices.astype(jnp.int32)
    iidx = item_indices.astype(jnp.int32)
    gran_u = embedding_user.T.reshape(LATENT, BLOCKS, LATENT)
    gran_i = embedding_item.T.reshape(LATENT, BLOCKS, LATENT)
    wvec = affine_W.reshape(LATENT)
    bvec = jnp.broadcast_to(affine_b.reshape(1), (LATENT,))

    mesh = plsc.VectorSubcoreMesh(core_axis_name="c", subcore_axis_name="s")
    run = functools.partial(
        pl.kernel,
        out_type=jax.ShapeDtypeStruct((BATCH,), jnp.float32),
        mesh=mesh,
        compiler_params=pltpu.CompilerParams(
            needs_layout_passes=False, use_tc_tiling_on_sc=False),
        scratch_types=[
            pltpu.VMEM((CHUNKS, CHUNK), jnp.int32),
            pltpu.VMEM((CHUNKS, CHUNK), jnp.int32),
            pltpu.VMEM((CHUNKS, CHUNK), jnp.int32),
            pltpu.VMEM((CHUNKS, CHUNK), jnp.int32),
            pltpu.VMEM((LATENT, CHUNK, LATENT), jnp.float32),
            pltpu.VMEM((LATENT, CHUNK, LATENT), jnp.float32),
            pltpu.VMEM((LATENT,), jnp.float32),
            pltpu.VMEM((LATENT,), jnp.float32),
            pltpu.VMEM((BPW,), jnp.float32),
            pltpu.SemaphoreType.DMA,
        ],
    )(_gmf_body)
    out = run(uidx, iidx, gran_u, gran_i, wvec, bvec)
    return out.reshape(BATCH, 1)


# speedup vs baseline: 3.1569x; 3.1569x over previous
import functools
import jax
import jax.numpy as jnp
from jax import lax
from jax.experimental import pallas as pl
from jax.experimental.pallas import tpu as pltpu
from jax.experimental.pallas import tpu_sc as plsc

BATCH = 16384
LATENT = 16
NUM_ROWS = 1000000
RPACK = 128 // LATENT
PACKED = NUM_ROWS // RPACK
NC = 2
NS = 16
NW = NC * NS
BPW = BATCH // NW
CHUNK = 128
CHUNKS = BPW // CHUNK
SUBGROUPS = CHUNK // LATENT


def _gmf_body(uidx_hbm, iidx_hbm, pu_hbm, pi_hbm, wvec_hbm, bvec_hbm, out_hbm,
              sidx_u, sidx_i, ublk_v, iblk_v, ubuf, ibuf,
              wvec_v, bvec_v, logits_v, sem):
    wid = lax.axis_index("s") * NC + lax.axis_index("c")
    base = wid * BPW

    for j in range(CHUNKS):
        pltpu.sync_copy(uidx_hbm.at[pl.ds(base + j * CHUNK, CHUNK)], sidx_u.at[j])
        pltpu.sync_copy(iidx_hbm.at[pl.ds(base + j * CHUNK, CHUNK)], sidx_i.at[j])
    pltpu.sync_copy(wvec_hbm, wvec_v)
    pltpu.sync_copy(bvec_hbm, bvec_v)

    iota = lax.iota(jnp.int32, LATENT)

    def blk_body(t, carry):
        j = t // SUBGROUPS
        s0 = (t % SUBGROUPS) * LATENT
        ublk_v[j, pl.ds(s0, LATENT)] = sidx_u[j, pl.ds(s0, LATENT)] >> 3
        iblk_v[j, pl.ds(s0, LATENT)] = sidx_i[j, pl.ds(s0, LATENT)] >> 3
        return carry

    lax.fori_loop(0, CHUNKS * SUBGROUPS, blk_body, 0)

    bv = bvec_v[...]
    wv = wvec_v[...]

    def chunk_body(j, carry):
        cu = pltpu.async_copy(pu_hbm.at[ublk_v.at[j]], ubuf, sem)
        ci = pltpu.async_copy(pi_hbm.at[iblk_v.at[j]], ibuf, sem)
        cu.wait()
        ci.wait()

        def group_body(g, carry2):
            s0 = g * LATENT
            lane_u = (sidx_u[j, pl.ds(s0, LATENT)] & 7) << 4
            lane_i = (sidx_i[j, pl.ds(s0, LATENT)] & 7) << 4
            rows16 = s0 + iota
            acc = bv
            for d in range(LATENT):
                u = plsc.load_gather(ubuf, [rows16, lane_u + d])
                i = plsc.load_gather(ibuf, [rows16, lane_i + d])
                acc = acc + u * i * wv[d]
            logits_v[pl.ds(j * CHUNK + s0, LATENT)] = 1.0 / (1.0 + jnp.exp(-acc))
            return carry2

        lax.fori_loop(0, SUBGROUPS, group_body, 0)
        return carry

    lax.fori_loop(0, CHUNKS, chunk_body, 0)

    pltpu.sync_copy(logits_v, out_hbm.at[pl.ds(base, BPW)])


def kernel(user_indices, item_indices, domain_idc, embedding_user,
           embedding_item, affine_W, affine_b):
    del domain_idc
    uidx = user_indices.astype(jnp.int32)
    iidx = item_indices.astype(jnp.int32)
    pu = embedding_user.reshape(PACKED, 128)
    pi = embedding_item.reshape(PACKED, 128)
    wvec = affine_W.reshape(LATENT)
    bvec = jnp.broadcast_to(affine_b.reshape(1), (LATENT,))

    mesh = plsc.VectorSubcoreMesh(core_axis_name="c", subcore_axis_name="s")
    run = functools.partial(
        pl.kernel,
        out_type=jax.ShapeDtypeStruct((BATCH,), jnp.float32),
        mesh=mesh,
        compiler_params=pltpu.CompilerParams(
            needs_layout_passes=False, use_tc_tiling_on_sc=True),
        scratch_types=[
            pltpu.VMEM((CHUNKS, CHUNK), jnp.int32),
            pltpu.VMEM((CHUNKS, CHUNK), jnp.int32),
            pltpu.VMEM((CHUNKS, CHUNK), jnp.int32),
            pltpu.VMEM((CHUNKS, CHUNK), jnp.int32),
            pltpu.VMEM((CHUNK, 128), jnp.float32),
            pltpu.VMEM((CHUNK, 128), jnp.float32),
            pltpu.VMEM((LATENT,), jnp.float32),
            pltpu.VMEM((LATENT,), jnp.float32),
            pltpu.VMEM((BPW,), jnp.float32),
            pltpu.SemaphoreType.DMA,
        ],
    )(_gmf_body)
    out = run(uidx, iidx, pu, pi, wvec, bvec)
    return out.reshape(BATCH, 1)


# speedup vs baseline: 12.1857x; 3.8600x over previous
import functools
import jax
import jax.numpy as jnp
from jax import lax
from jax.experimental import pallas as pl
from jax.experimental.pallas import tpu as pltpu
from jax.experimental.pallas import tpu_sc as plsc

BATCH = 16384
LATENT = 16
NUM_ROWS = 1000000
TCOLS = 7813
PACKED = TCOLS * LATENT
GRANULES = PACKED * 8
NC = 2
NS = 16
NW = NC * NS
BPW = BATCH // NW
CHUNK = 128
CHUNKS = BPW // CHUNK
SUBGROUPS = CHUNK // LATENT
WAVE = 4
STEPS = 245
PAIRS = (STEPS + 2 * WAVE - 1) // (2 * WAVE)


def _repack_body(src_hbm, dst_hbm, bufs, rsem, wsem):
    wid = lax.axis_index("s") * NC + lax.axis_index("c")

    def _drain_write(b):
        pltpu.make_async_copy(
            bufs.at[b], dst_hbm.at[pl.ds(0, LATENT), :], wsem).wait()

    def _drain_read(b):
        pltpu.make_async_copy(
            src_hbm.at[:, pl.ds(0, 128)], bufs.at[b], rsem).wait()

    def pair_body(k, carry):
        for half in range(2):
            ts = [(2 * k + half) * WAVE + s for s in range(WAVE)]
            blks = [wid + 32 * t for t in ts]
            valids = [blks[s] < TCOLS - 1 for s in range(WAVE)]
            for s in range(WAVE):
                pl.when(valids[s] & (ts[s] >= 2 * WAVE))(
                    lambda b=half * WAVE + s: _drain_write(b))
            for s in range(WAVE):
                def _rd(s=s, half=half):
                    pltpu.async_copy(
                        src_hbm.at[:, pl.ds(pl.multiple_of(blks[s] * 128, 128),
                                            128)],
                        bufs.at[half * WAVE + s], rsem)
                pl.when(valids[s])(_rd)
            for s in range(WAVE):
                pl.when(valids[s])(
                    lambda b=half * WAVE + s: _drain_read(b))
            for s in range(WAVE):
                def _wr(s=s, half=half):
                    pltpu.async_copy(
                        bufs.at[half * WAVE + s],
                        dst_hbm.at[pl.ds(pl.multiple_of(blks[s] * LATENT, 16),
                                         LATENT), :],
                        wsem)
                pl.when(valids[s])(_wr)
        return carry

    lax.fori_loop(0, PAIRS, pair_body, 0)

    for b in range(2 * WAVE):
        _drain_write(b)


TAIL0 = (TCOLS - 1) * 128


def _gather_body(uidx_hbm, iidx_hbm, gu_hbm, gi_hbm, tu_hbm, ti_hbm,
                 wvec_hbm, bvec_hbm, out_hbm, sidx_u, sidx_i, uq_v, iq_v,
                 ubuf, ibuf, tu_v, ti_v, wvec_v, bvec_v, logits_v, sem):
    wid = lax.axis_index("s") * NC + lax.axis_index("c")
    base = wid * BPW

    for j in range(CHUNKS):
        pltpu.sync_copy(uidx_hbm.at[pl.ds(base + j * CHUNK, CHUNK)], sidx_u.at[j])
        pltpu.sync_copy(iidx_hbm.at[pl.ds(base + j * CHUNK, CHUNK)], sidx_i.at[j])
    pltpu.sync_copy(wvec_hbm, wvec_v)
    pltpu.sync_copy(bvec_hbm, bvec_v)
    pltpu.sync_copy(tu_hbm, tu_v)
    pltpu.sync_copy(ti_hbm, ti_v)

    iota = lax.iota(jnp.int32, LATENT)
    bv = bvec_v[...]
    wv = wvec_v[...]
    dsplats = [jnp.full((LATENT,), d, jnp.int32) for d in range(LATENT)]

    def chunk_body(j, carry):
        def idx_body(g, carry2):
            s0 = g * LATENT
            su = jnp.minimum(sidx_u[j, pl.ds(s0, LATENT)], TAIL0 - 1)
            si = jnp.minimum(sidx_i[j, pl.ds(s0, LATENT)], TAIL0 - 1)
            bu = ((su >> 7) << 7) + ((su >> 4) & 7)
            bi = ((si >> 7) << 7) + ((si >> 4) & 7)
            for d in range(LATENT):
                uq_v[d, pl.ds(s0, LATENT)] = bu + 8 * d
                iq_v[d, pl.ds(s0, LATENT)] = bi + 8 * d
            return carry2

        lax.fori_loop(0, SUBGROUPS, idx_body, 0)

        copies = []
        for d in range(LATENT):
            copies.append(pltpu.async_copy(
                gu_hbm.at[uq_v.at[d]], ubuf.at[d], sem))
            copies.append(pltpu.async_copy(
                gi_hbm.at[iq_v.at[d]], ibuf.at[d], sem))
        for c in copies:
            c.wait()

        def group_body(g, carry2):
            s0 = g * LATENT
            su = sidx_u[j, pl.ds(s0, LATENT)]
            si = sidx_i[j, pl.ds(s0, LATENT)]
            lane_u = jnp.minimum(su, TAIL0 - 1) & 15
            lane_i = jnp.minimum(si, TAIL0 - 1) & 15
            rows16 = s0 + iota
            acc = bv
            for d in range(LATENT):
                u = plsc.load_gather(ubuf, [dsplats[d], rows16, lane_u])
                i = plsc.load_gather(ibuf, [dsplats[d], rows16, lane_i])
                acc = acc + u * i * wv[d]
            logits_v[pl.ds(j * CHUNK + s0, LATENT)] = 1.0 / (1.0 + jnp.exp(-acc))

            mask_u = su >= TAIL0
            mask_i = si >= TAIL0

            def _patch():
                tru = jnp.minimum(jnp.maximum(su - TAIL0, 0), 63)
                tri = jnp.minimum(jnp.maximum(si - TAIL0, 0), 63)
                acc2 = bv
                for d in range(LATENT):
                    u = plsc.load_gather(ubuf, [dsplats[d], rows16, lane_u])
                    i = plsc.load_gather(ibuf, [dsplats[d], rows16, lane_i])
                    ut = plsc.load_gather(tu_v, [tru, dsplats[d]])
                    it = plsc.load_gather(ti_v, [tri, dsplats[d]])
                    u = jnp.where(mask_u, ut, u)
                    i = jnp.where(mask_i, it, i)
                    acc2 = acc2 + u * i * wv[d]
                logits_v[pl.ds(j * CHUNK + s0, LATENT)] = (
                    1.0 / (1.0 + jnp.exp(-acc2)))

            pl.when(jnp.any(mask_u | mask_i))(_patch)
            return carry2

        lax.fori_loop(0, SUBGROUPS, group_body, 0)
        return carry

    lax.fori_loop(0, CHUNKS, chunk_body, 0)

    pltpu.sync_copy(logits_v, out_hbm.at[pl.ds(base, BPW)])


def kernel(user_indices, item_indices, domain_idc, embedding_user,
           embedding_item, affine_W, affine_b):
    del domain_idc
    uidx = user_indices.astype(jnp.int32)
    iidx = item_indices.astype(jnp.int32)
    eut = embedding_user.T
    eit = embedding_item.T
    wvec = affine_W.reshape(LATENT)
    bvec = jnp.broadcast_to(affine_b.reshape(1), (LATENT,))

    mesh = plsc.VectorSubcoreMesh(core_axis_name="c", subcore_axis_name="s")

    repack = functools.partial(
        pl.kernel,
        out_type=jax.ShapeDtypeStruct((PACKED, 128), jnp.float32),
        mesh=mesh,
        compiler_params=pltpu.CompilerParams(
            needs_layout_passes=False, use_tc_tiling_on_sc=True),
        scratch_types=[
            pltpu.VMEM((2 * WAVE, LATENT, 128), jnp.float32),
            pltpu.SemaphoreType.DMA,
            pltpu.SemaphoreType.DMA,
        ],
    )(_repack_body)
    pu = repack(eut)
    pi = repack(eit)

    gather = functools.partial(
        pl.kernel,
        out_type=jax.ShapeDtypeStruct((BATCH,), jnp.float32),
        mesh=mesh,
        compiler_params=pltpu.CompilerParams(
            needs_layout_passes=False, use_tc_tiling_on_sc=False),
        scratch_types=[
            pltpu.VMEM((CHUNKS, CHUNK), jnp.int32),
            pltpu.VMEM((CHUNKS, CHUNK), jnp.int32),
            pltpu.VMEM((LATENT, CHUNK), jnp.int32),
            pltpu.VMEM((LATENT, CHUNK), jnp.int32),
            pltpu.VMEM((LATENT, CHUNK, LATENT), jnp.float32),
            pltpu.VMEM((LATENT, CHUNK, LATENT), jnp.float32),
            pltpu.VMEM((64, LATENT), jnp.float32),
            pltpu.VMEM((64, LATENT), jnp.float32),
            pltpu.VMEM((LATENT,), jnp.float32),
            pltpu.VMEM((LATENT,), jnp.float32),
            pltpu.VMEM((BPW,), jnp.float32),
            pltpu.SemaphoreType.DMA,
        ],
    )(_gather_body)
    tail_u = embedding_user[TAIL0:, :]
    tail_i = embedding_item[TAIL0:, :]
    out = gather(uidx, iidx,
                 pu.reshape(GRANULES, LATENT), pi.reshape(GRANULES, LATENT),
                 tail_u, tail_i, wvec, bvec)
    return out.reshape(BATCH, 1)


# speedup vs baseline: 16.4213x; 1.3476x over previous
import functools
import jax
import jax.numpy as jnp
from jax import lax
from jax.experimental import pallas as pl
from jax.experimental.pallas import tpu as pltpu
from jax.experimental.pallas import tpu_sc as plsc

BATCH = 16384
LATENT = 16
NUM_ROWS = 1000000
TCOLS = 7813
PACKED = TCOLS * LATENT
GRANULES = PACKED * 8
NC = 2
NS = 16
NW = NC * NS
BPW = BATCH // NW
CHUNK = 128
CHUNKS = BPW // CHUNK
SUBGROUPS = CHUNK // LATENT
WAVE = 4
STEPS = 245
PAIRS = (STEPS + 2 * WAVE - 1) // (2 * WAVE)


GROUPW = 4
CGRPS = (TCOLS - 1) // GROUPW
RSTEPS = (CGRPS + NW - 1) // NW
RPAIRS = (RSTEPS + 2 * WAVE - 1) // (2 * WAVE)


def _pipe_one_table(wid, src_hbm, dst_hbm, bufs, rsem, wsem):
    def _drain_write(b, k):
        pltpu.make_async_copy(
            bufs.at[b, :, pl.ds(k * 128, 128)],
            dst_hbm.at[pl.ds(0, LATENT), :], wsem).wait()

    def _drain_read(b):
        pltpu.make_async_copy(
            src_hbm.at[:, pl.ds(0, GROUPW * 128)], bufs.at[b], rsem).wait()

    def pair_body(k, carry):
        for half in range(2):
            ts = [(2 * k + half) * WAVE + s for s in range(WAVE)]
            cgs = [wid + NW * t for t in ts]
            valids = [cgs[s] < CGRPS for s in range(WAVE)]
            for s in range(WAVE):
                def _dw(b=half * WAVE + s):
                    for q in range(GROUPW):
                        _drain_write(b, q)
                pl.when(valids[s] & (ts[s] >= 2 * WAVE))(_dw)
            for s in range(WAVE):
                def _rd(s=s, half=half):
                    pltpu.async_copy(
                        src_hbm.at[:, pl.ds(
                            pl.multiple_of(cgs[s] * (GROUPW * 128), 128),
                            GROUPW * 128)],
                        bufs.at[half * WAVE + s], rsem)
                pl.when(valids[s])(_rd)
            for s in range(WAVE):
                pl.when(valids[s])(
                    lambda b=half * WAVE + s: _drain_read(b))
            for s in range(WAVE):
                def _wr(s=s, half=half):
                    for q in range(GROUPW):
                        pltpu.async_copy(
                            bufs.at[half * WAVE + s, :, pl.ds(q * 128, 128)],
                            dst_hbm.at[pl.ds(pl.multiple_of(
                                (cgs[s] * GROUPW + q) * LATENT, 16),
                                LATENT), :],
                            wsem)
                pl.when(valids[s])(_wr)
        return carry

    lax.fori_loop(0, RPAIRS, pair_body, 0)

    for b in range(2 * WAVE):
        for q in range(GROUPW):
            _drain_write(b, q)


def _repack_body(srcu_hbm, srci_hbm, dstu_hbm, dsti_hbm, bufs, rsem, wsem):
    wid = lax.axis_index("s") * NC + lax.axis_index("c")
    _pipe_one_table(wid, srcu_hbm, dstu_hbm, bufs, rsem, wsem)
    _pipe_one_table(wid, srci_hbm, dsti_hbm, bufs, rsem, wsem)


TAIL0 = (TCOLS - 1) * 128


def _gather_body(uidx_hbm, iidx_hbm, gu_hbm, gi_hbm, tu_hbm, ti_hbm,
                 wvec_hbm, bvec_hbm, out_hbm, sidx_u, sidx_i, uq_v, iq_v,
                 ubuf, ibuf, tu_v, ti_v, wvec_v, bvec_v, logits_v, sem):
    wid = lax.axis_index("s") * NC + lax.axis_index("c")
    base = wid * BPW

    for j in range(CHUNKS):
        pltpu.sync_copy(uidx_hbm.at[pl.ds(base + j * CHUNK, CHUNK)], sidx_u.at[j])
        pltpu.sync_copy(iidx_hbm.at[pl.ds(base + j * CHUNK, CHUNK)], sidx_i.at[j])
    pltpu.sync_copy(wvec_hbm, wvec_v)
    pltpu.sync_copy(bvec_hbm, bvec_v)
    pltpu.sync_copy(tu_hbm, tu_v)
    pltpu.sync_copy(ti_hbm, ti_v)

    iota = lax.iota(jnp.int32, LATENT)
    bv = bvec_v[...]
    wv = wvec_v[...]
    dsplats = [jnp.full((LATENT,), d, jnp.int32) for d in range(LATENT)]

    def chunk_body(j, carry):
        def idx_body(g, carry2):
            s0 = g * LATENT
            su = jnp.minimum(sidx_u[j, pl.ds(s0, LATENT)], TAIL0 - 1)
            si = jnp.minimum(sidx_i[j, pl.ds(s0, LATENT)], TAIL0 - 1)
            bu = ((su >> 7) << 7) + ((su >> 4) & 7)
            bi = ((si >> 7) << 7) + ((si >> 4) & 7)
            for d in range(LATENT):
                uq_v[d, pl.ds(s0, LATENT)] = bu + 8 * d
                iq_v[d, pl.ds(s0, LATENT)] = bi + 8 * d
            return carry2

        lax.fori_loop(0, SUBGROUPS, idx_body, 0)

        copies = []
        for d in range(LATENT):
            copies.append(pltpu.async_copy(
                gu_hbm.at[uq_v.at[d]], ubuf.at[d], sem))
            copies.append(pltpu.async_copy(
                gi_hbm.at[iq_v.at[d]], ibuf.at[d], sem))
        for c in copies:
            c.wait()

        def group_body(g, carry2):
            s0 = g * LATENT
            su = sidx_u[j, pl.ds(s0, LATENT)]
            si = sidx_i[j, pl.ds(s0, LATENT)]
            lane_u = jnp.minimum(su, TAIL0 - 1) & 15
            lane_i = jnp.minimum(si, TAIL0 - 1) & 15
            rows16 = s0 + iota
            acc = bv
            for d in range(LATENT):
                u = plsc.load_gather(ubuf, [dsplats[d], rows16, lane_u])
                i = plsc.load_gather(ibuf, [dsplats[d], rows16, lane_i])
                acc = acc + u * i * wv[d]
            logits_v[pl.ds(j * CHUNK + s0, LATENT)] = 1.0 / (1.0 + jnp.exp(-acc))

            mask_u = su >= TAIL0
            mask_i = si >= TAIL0

            def _patch():
                tru = jnp.minimum(jnp.maximum(su - TAIL0, 0), 63)
                tri = jnp.minimum(jnp.maximum(si - TAIL0, 0), 63)
                acc2 = bv
                for d in range(LATENT):
                    u = plsc.load_gather(ubuf, [dsplats[d], rows16, lane_u])
                    i = plsc.load_gather(ibuf, [dsplats[d], rows16, lane_i])
                    ut = plsc.load_gather(tu_v, [tru, dsplats[d]])
                    it = plsc.load_gather(ti_v, [tri, dsplats[d]])
                    u = jnp.where(mask_u, ut, u)
                    i = jnp.where(mask_i, it, i)
                    acc2 = acc2 + u * i * wv[d]
                logits_v[pl.ds(j * CHUNK + s0, LATENT)] = (
                    1.0 / (1.0 + jnp.exp(-acc2)))

            pl.when(jnp.any(mask_u | mask_i))(_patch)
            return carry2

        lax.fori_loop(0, SUBGROUPS, group_body, 0)
        return carry

    lax.fori_loop(0, CHUNKS, chunk_body, 0)

    pltpu.sync_copy(logits_v, out_hbm.at[pl.ds(base, BPW)])


def kernel(user_indices, item_indices, domain_idc, embedding_user,
           embedding_item, affine_W, affine_b):
    del domain_idc
    uidx = user_indices.astype(jnp.int32)
    iidx = item_indices.astype(jnp.int32)
    eut = embedding_user.T
    eit = embedding_item.T
    wvec = affine_W.reshape(LATENT)
    bvec = jnp.broadcast_to(affine_b.reshape(1), (LATENT,))

    mesh = plsc.VectorSubcoreMesh(core_axis_name="c", subcore_axis_name="s")

    repack = functools.partial(
        pl.kernel,
        out_type=(jax.ShapeDtypeStruct((PACKED, 128), jnp.float32),
                  jax.ShapeDtypeStruct((PACKED, 128), jnp.float32)),
        mesh=mesh,
        compiler_params=pltpu.CompilerParams(
            needs_layout_passes=False, use_tc_tiling_on_sc=True),
        scratch_types=[
            pltpu.VMEM((2 * WAVE, LATENT, GROUPW * 128), jnp.float32),
            pltpu.SemaphoreType.DMA,
            pltpu.SemaphoreType.DMA,
        ],
    )(_repack_body)
    pu, pi = repack(eut, eit)

    gather = functools.partial(
        pl.kernel,
        out_type=jax.ShapeDtypeStruct((BATCH,), jnp.float32),
        mesh=mesh,
        compiler_params=pltpu.CompilerParams(
            needs_layout_passes=False, use_tc_tiling_on_sc=False),
        scratch_types=[
            pltpu.VMEM((CHUNKS, CHUNK), jnp.int32),
            pltpu.VMEM((CHUNKS, CHUNK), jnp.int32),
            pltpu.VMEM((LATENT, CHUNK), jnp.int32),
            pltpu.VMEM((LATENT, CHUNK), jnp.int32),
            pltpu.VMEM((LATENT, CHUNK, LATENT), jnp.float32),
            pltpu.VMEM((LATENT, CHUNK, LATENT), jnp.float32),
            pltpu.VMEM((64, LATENT), jnp.float32),
            pltpu.VMEM((64, LATENT), jnp.float32),
            pltpu.VMEM((LATENT,), jnp.float32),
            pltpu.VMEM((LATENT,), jnp.float32),
            pltpu.VMEM((BPW,), jnp.float32),
            pltpu.SemaphoreType.DMA,
        ],
    )(_gather_body)
    tail_u = embedding_user[TAIL0:, :]
    tail_i = embedding_item[TAIL0:, :]
    out = gather(uidx, iidx,
                 pu.reshape(GRANULES, LATENT), pi.reshape(GRANULES, LATENT),
                 tail_u, tail_i, wvec, bvec)
    return out.reshape(BATCH, 1)


# speedup vs baseline: 16.6334x; 1.0129x over previous
import functools
import jax
import jax.numpy as jnp
from jax import lax
from jax.experimental import pallas as pl
from jax.experimental.pallas import tpu as pltpu
from jax.experimental.pallas import tpu_sc as plsc

BATCH = 16384
LATENT = 16
NUM_ROWS = 1000000
TCOLS = 7813
PACKED = TCOLS * LATENT
GRANULES = PACKED * 8
NC = 2
NS = 16
NW = NC * NS
BPW = BATCH // NW
CHUNK = 128
CHUNKS = BPW // CHUNK
SUBGROUPS = CHUNK // LATENT
WAVE = 2
STEPS = 245
PAIRS = (STEPS + 2 * WAVE - 1) // (2 * WAVE)


GROUPW = 6
CGRPS = (TCOLS - 1) // GROUPW
RSTEPS = (CGRPS + NW - 1) // NW
RPAIRS = (RSTEPS + 2 * WAVE - 1) // (2 * WAVE)


def _pipe_one_table(wid, src_hbm, dst_hbm, bufs, rsem, wsem):
    def _drain_write(b, k):
        pltpu.make_async_copy(
            bufs.at[b, :, pl.ds(k * 128, 128)],
            dst_hbm.at[pl.ds(0, LATENT), :], wsem).wait()

    def _drain_read(b):
        pltpu.make_async_copy(
            src_hbm.at[:, pl.ds(0, GROUPW * 128)], bufs.at[b], rsem).wait()

    def pair_body(k, carry):
        for half in range(2):
            ts = [(2 * k + half) * WAVE + s for s in range(WAVE)]
            cgs = [wid + NW * t for t in ts]
            valids = [cgs[s] < CGRPS for s in range(WAVE)]
            for s in range(WAVE):
                def _dw(b=half * WAVE + s):
                    for q in range(GROUPW):
                        _drain_write(b, q)
                pl.when(valids[s] & (ts[s] >= 2 * WAVE))(_dw)
            for s in range(WAVE):
                def _rd(s=s, half=half):
                    pltpu.async_copy(
                        src_hbm.at[:, pl.ds(
                            pl.multiple_of(cgs[s] * (GROUPW * 128), 128),
                            GROUPW * 128)],
                        bufs.at[half * WAVE + s], rsem)
                pl.when(valids[s])(_rd)
            for s in range(WAVE):
                pl.when(valids[s])(
                    lambda b=half * WAVE + s: _drain_read(b))
            for s in range(WAVE):
                def _wr(s=s, half=half):
                    for q in range(GROUPW):
                        pltpu.async_copy(
                            bufs.at[half * WAVE + s, :, pl.ds(q * 128, 128)],
                            dst_hbm.at[pl.ds(pl.multiple_of(
                                (cgs[s] * GROUPW + q) * LATENT, 16),
                                LATENT), :],
                            wsem)
                pl.when(valids[s])(_wr)
        return carry

    lax.fori_loop(0, RPAIRS, pair_body, 0)

    for b in range(2 * WAVE):
        for q in range(GROUPW):
            _drain_write(b, q)


def _repack_body(srcu_hbm, srci_hbm, dstu_hbm, dsti_hbm, bufs, rsem, wsem):
    wid = lax.axis_index("s") * NC + lax.axis_index("c")
    _pipe_one_table(wid, srcu_hbm, dstu_hbm, bufs, rsem, wsem)
    _pipe_one_table(wid, srci_hbm, dsti_hbm, bufs, rsem, wsem)


TAIL0 = (TCOLS - 1) * 128


def _gather_body(uidx_hbm, iidx_hbm, gu_hbm, gi_hbm, tu_hbm, ti_hbm,
                 wvec_hbm, bvec_hbm, out_hbm, sidx_u, sidx_i, uq_v, iq_v,
                 ubuf, ibuf, tu_v, ti_v, wvec_v, bvec_v, logits_v, sem):
    wid = lax.axis_index("s") * NC + lax.axis_index("c")
    base = wid * BPW

    for j in range(CHUNKS):
        pltpu.sync_copy(uidx_hbm.at[pl.ds(base + j * CHUNK, CHUNK)], sidx_u.at[j])
        pltpu.sync_copy(iidx_hbm.at[pl.ds(base + j * CHUNK, CHUNK)], sidx_i.at[j])
    pltpu.sync_copy(wvec_hbm, wvec_v)
    pltpu.sync_copy(bvec_hbm, bvec_v)
    pltpu.sync_copy(tu_hbm, tu_v)
    pltpu.sync_copy(ti_hbm, ti_v)

    iota = lax.iota(jnp.int32, LATENT)
    bv = bvec_v[...]
    wv = wvec_v[...]
    dsplats = [jnp.full((LATENT,), d, jnp.int32) for d in range(LATENT)]

    def chunk_body(j, carry):
        def idx_body(g, carry2):
            s0 = g * LATENT
            su = jnp.minimum(sidx_u[j, pl.ds(s0, LATENT)], TAIL0 - 1)
            si = jnp.minimum(sidx_i[j, pl.ds(s0, LATENT)], TAIL0 - 1)
            bu = ((su >> 7) << 7) + ((su >> 4) & 7)
            bi = ((si >> 7) << 7) + ((si >> 4) & 7)
            for d in range(LATENT):
                uq_v[d, pl.ds(s0, LATENT)] = bu + 8 * d
                iq_v[d, pl.ds(s0, LATENT)] = bi + 8 * d
            return carry2

        lax.fori_loop(0, SUBGROUPS, idx_body, 0)

        copies = []
        for d in range(LATENT):
            copies.append(pltpu.async_copy(
                gu_hbm.at[uq_v.at[d]], ubuf.at[d], sem))
            copies.append(pltpu.async_copy(
                gi_hbm.at[iq_v.at[d]], ibuf.at[d], sem))
        for c in copies:
            c.wait()

        def group_body(g, carry2):
            s0 = g * LATENT
            su = sidx_u[j, pl.ds(s0, LATENT)]
            si = sidx_i[j, pl.ds(s0, LATENT)]
            lane_u = jnp.minimum(su, TAIL0 - 1) & 15
            lane_i = jnp.minimum(si, TAIL0 - 1) & 15
            rows16 = s0 + iota
            acc = bv
            for d in range(LATENT):
                u = plsc.load_gather(ubuf, [dsplats[d], rows16, lane_u])
                i = plsc.load_gather(ibuf, [dsplats[d], rows16, lane_i])
                acc = acc + u * i * wv[d]
            logits_v[pl.ds(j * CHUNK + s0, LATENT)] = 1.0 / (1.0 + jnp.exp(-acc))

            mask_u = su >= TAIL0
            mask_i = si >= TAIL0

            def _patch():
                tru = jnp.minimum(jnp.maximum(su - TAIL0, 0), 63)
                tri = jnp.minimum(jnp.maximum(si - TAIL0, 0), 63)
                acc2 = bv
                for d in range(LATENT):
                    u = plsc.load_gather(ubuf, [dsplats[d], rows16, lane_u])
                    i = plsc.load_gather(ibuf, [dsplats[d], rows16, lane_i])
                    ut = plsc.load_gather(tu_v, [tru, dsplats[d]])
                    it = plsc.load_gather(ti_v, [tri, dsplats[d]])
                    u = jnp.where(mask_u, ut, u)
                    i = jnp.where(mask_i, it, i)
                    acc2 = acc2 + u * i * wv[d]
                logits_v[pl.ds(j * CHUNK + s0, LATENT)] = (
                    1.0 / (1.0 + jnp.exp(-acc2)))

            pl.when(jnp.any(mask_u | mask_i))(_patch)
            return carry2

        lax.fori_loop(0, SUBGROUPS, group_body, 0)
        return carry

    lax.fori_loop(0, CHUNKS, chunk_body, 0)

    pltpu.sync_copy(logits_v, out_hbm.at[pl.ds(base, BPW)])


def kernel(user_indices, item_indices, domain_idc, embedding_user,
           embedding_item, affine_W, affine_b):
    del domain_idc
    uidx = user_indices.astype(jnp.int32)
    iidx = item_indices.astype(jnp.int32)
    eut = embedding_user.T
    eit = embedding_item.T
    wvec = affine_W.reshape(LATENT)
    bvec = jnp.broadcast_to(affine_b.reshape(1), (LATENT,))

    mesh = plsc.VectorSubcoreMesh(core_axis_name="c", subcore_axis_name="s")

    repack = functools.partial(
        pl.kernel,
        out_type=(jax.ShapeDtypeStruct((PACKED, 128), jnp.float32),
                  jax.ShapeDtypeStruct((PACKED, 128), jnp.float32)),
        mesh=mesh,
        compiler_params=pltpu.CompilerParams(
            needs_layout_passes=False, use_tc_tiling_on_sc=True),
        scratch_types=[
            pltpu.VMEM((2 * WAVE, LATENT, GROUPW * 128), jnp.float32),
            pltpu.SemaphoreType.DMA,
            pltpu.SemaphoreType.DMA,
        ],
    )(_repack_body)
    pu, pi = repack(eut, eit)

    gather = functools.partial(
        pl.kernel,
        out_type=jax.ShapeDtypeStruct((BATCH,), jnp.float32),
        mesh=mesh,
        compiler_params=pltpu.CompilerParams(
            needs_layout_passes=False, use_tc_tiling_on_sc=False),
        scratch_types=[
            pltpu.VMEM((CHUNKS, CHUNK), jnp.int32),
            pltpu.VMEM((CHUNKS, CHUNK), jnp.int32),
            pltpu.VMEM((LATENT, CHUNK), jnp.int32),
            pltpu.VMEM((LATENT, CHUNK), jnp.int32),
            pltpu.VMEM((LATENT, CHUNK, LATENT), jnp.float32),
            pltpu.VMEM((LATENT, CHUNK, LATENT), jnp.float32),
            pltpu.VMEM((64, LATENT), jnp.float32),
            pltpu.VMEM((64, LATENT), jnp.float32),
            pltpu.VMEM((LATENT,), jnp.float32),
            pltpu.VMEM((LATENT,), jnp.float32),
            pltpu.VMEM((BPW,), jnp.float32),
            pltpu.SemaphoreType.DMA,
        ],
    )(_gather_body)
    tail_u = embedding_user[TAIL0:, :]
    tail_i = embedding_item[TAIL0:, :]
    out = gather(uidx, iidx,
                 pu.reshape(GRANULES, LATENT), pi.reshape(GRANULES, LATENT),
                 tail_u, tail_i, wvec, bvec)
    return out.reshape(BATCH, 1)
